# baseline (speedup 1.0000x reference)
import functools

import jax
import jax.numpy as jnp
from jax import lax
from jax.experimental import pallas as pl
from jax.experimental.pallas import tpu as pltpu
from jax.experimental.pallas import tpu_sc as plsc

DIM = 64
B = 16384
B_T = 1.0
B_R = 2.0

NC = 2
NS = 16
L = 16
NW = NC * NS
BPW = B // NW
C = 256
G = C // L


def _rsqrt(x):
    i = plsc.bitcast(x, jnp.int32)
    i = jnp.int32(0x5F3759DF) - lax.shift_right_logical(i, 1)
    y = plsc.bitcast(i, jnp.float32)
    for _ in range(3):
        y = y * (1.5 - 0.5 * x * y * y)
    return y


def _maxnorm_scale(ss):
    n = ss * _rsqrt(ss)
    return jnp.minimum(1.0, 1.0 / (n + 1e-7))


def _iota16():
    return lax.iota(jnp.int32, 16)


def _bin_chunk(rows_i, rows_c, c0, c1, acc):
    zero = jnp.zeros((L,), jnp.float32)

    def group(g, acc):
        rowv = g * L + _iota16()

        def dloop(d, carry):
            f, ss = carry
            col = jnp.full((L,), d, jnp.int32)
            iv = plsc.load_gather(rows_i, [rowv, col])
            cv = plsc.load_gather(rows_c, [rowv, col])
            return f + iv * cv, ss + iv * iv

        f, ss = lax.fori_loop(0, DIM, dloop, (zero, zero))
        fs = f * _maxnorm_scale(ss)
        return acc + jnp.maximum(c0 + c1 * fs, 0.0)

    return lax.fori_loop(0, G, group, acc)


def _tri_chunk(rows_h, rows_t, rows_r, c0, c1, acc):
    zero = jnp.zeros((L,), jnp.float32)

    def group(g, acc):
        rowv = g * L + _iota16()

        def dloop(d, carry):
            f1, f2, sh, st, sr = carry
            col = jnp.full((L,), d, jnp.int32)
            col2 = jnp.full((L,), d + DIM, jnp.int32)
            hv = plsc.load_gather(rows_h, [rowv, col])
            tv = plsc.load_gather(rows_t, [rowv, col])
            r1 = plsc.load_gather(rows_r, [rowv, col])
            r2 = plsc.load_gather(rows_r, [rowv, col2])
            return (f1 + hv * r1, f2 + tv * r2,
                    sh + hv * hv, st + tv * tv, sr + r1 * r1 + r2 * r2)

        f1, f2, sh, st, sr = lax.fori_loop(
            0, DIM, dloop, (zero, zero, zero, zero, zero))
        fs = _maxnorm_scale(sr) * (_maxnorm_scale(sh) * f1
                                   + _maxnorm_scale(st) * f2)
        return acc + jnp.maximum(c0 + c1 * fs, 0.0)

    return lax.fori_loop(0, G, group, acc)


@functools.partial(
    pl.kernel,
    out_type=jax.ShapeDtypeStruct((NW, L), jnp.float32),
    mesh=plsc.VectorSubcoreMesh(core_axis_name="c", subcore_axis_name="s"),
    compiler_params=pltpu.CompilerParams(
        use_tc_tiling_on_sc=False, needs_layout_passes=False),
    scratch_types=[
        pltpu.VMEM((C,), jnp.int32),
        pltpu.VMEM((C,), jnp.int32),
        pltpu.VMEM((C,), jnp.int32),
        pltpu.VMEM((C, DIM), jnp.float32),
        pltpu.VMEM((C, DIM), jnp.float32),
        pltpu.VMEM((C, 2 * DIM), jnp.float32),
        pltpu.VMEM((L,), jnp.float32),
        pltpu.SemaphoreType.DMA,
    ],
)
def _sc_loss(inst_pos, con_pos, inst_neg, con_neg, head_pos, tail_pos,
             rel_pos, head_neg, tail_neg, rel_neg,
             instance_emb, concept_emb, rel_emb, out_hbm,
             idx_a, idx_b, idx_c, rows_a, rows_b, rows_r, part_v, sem):
    wid = lax.axis_index("s") * NC + lax.axis_index("c")
    base = wid * BPW
    acc = jnp.zeros((L,), jnp.float32)
    for ch in range(BPW // C):
        cb = base + ch * C

        pltpu.sync_copy(inst_pos.at[pl.ds(cb, C)], idx_a)
        pltpu.sync_copy(con_pos.at[pl.ds(cb, C)], idx_b)
        pltpu.async_copy(instance_emb.at[idx_a], rows_a, sem).wait()
        pltpu.async_copy(concept_emb.at[idx_b], rows_b, sem).wait()
        acc = _bin_chunk(rows_a, rows_b, B_T, -1.0, acc)

        pltpu.sync_copy(inst_neg.at[pl.ds(cb, C)], idx_a)
        pltpu.sync_copy(con_neg.at[pl.ds(cb, C)], idx_b)
        pltpu.async_copy(instance_emb.at[idx_a], rows_a, sem).wait()
        pltpu.async_copy(concept_emb.at[idx_b], rows_b, sem).wait()
        acc = _bin_chunk(rows_a, rows_b, -B_T, 1.0, acc)

        pltpu.sync_copy(head_pos.at[pl.ds(cb, C)], idx_a)
        pltpu.sync_copy(tail_pos.at[pl.ds(cb, C)], idx_b)
        pltpu.sync_copy(rel_pos.at[pl.ds(cb, C)], idx_c)
        pltpu.async_copy(instance_emb.at[idx_a], rows_a, sem).wait()
        pltpu.async_copy(instance_emb.at[idx_b], rows_b, sem).wait()
        pltpu.async_copy(rel_emb.at[idx_c], rows_r, sem).wait()
        acc = _tri_chunk(rows_a, rows_b, rows_r, B_R, -1.0, acc)

        pltpu.sync_copy(head_neg.at[pl.ds(cb, C)], idx_a)
        pltpu.sync_copy(tail_neg.at[pl.ds(cb, C)], idx_b)
        pltpu.sync_copy(rel_neg.at[pl.ds(cb, C)], idx_c)
        pltpu.async_copy(instance_emb.at[idx_a], rows_a, sem).wait()
        pltpu.async_copy(instance_emb.at[idx_b], rows_b, sem).wait()
        pltpu.async_copy(rel_emb.at[idx_c], rows_r, sem).wait()
        acc = _tri_chunk(rows_a, rows_b, rows_r, -B_R, 1.0, acc)

    part_v[...] = acc
    pltpu.sync_copy(part_v, out_hbm.at[wid])


def _sum_body(x_ref, o_ref):
    o_ref[0, 0] = jnp.sum(x_ref[...])


def _final_sum(partials):
    return pl.pallas_call(
        _sum_body,
        out_shape=jax.ShapeDtypeStruct((1, 1), jnp.float32),
        in_specs=[pl.BlockSpec(memory_space=pltpu.VMEM)],
        out_specs=pl.BlockSpec(memory_space=pltpu.SMEM),
    )(partials)


def kernel(inst_pos, con_pos, inst_neg, con_neg, head_pos, tail_pos,
           rel_pos, head_neg, tail_neg, rel_neg,
           instance_emb, concept_emb, rel_emb):
    partials = _sc_loss(inst_pos, con_pos, inst_neg, con_neg,
                        head_pos, tail_pos, rel_pos,
                        head_neg, tail_neg, rel_neg,
                        instance_emb, concept_emb, rel_emb)
    return _final_sum(partials)[0, 0]


# speedup vs baseline: 1.2594x; 1.2594x over previous
import functools

import jax
import jax.numpy as jnp
from jax import lax
from jax.experimental import pallas as pl
from jax.experimental.pallas import tpu as pltpu
from jax.experimental.pallas import tpu_sc as plsc

DIM = 64
B = 16384
B_T = 1.0
B_R = 2.0

NC = 2
NS = 16
L = 16
NW = NC * NS
BPW = B // NW
C = 256
G = C // L
NCH = BPW // C


def _rsqrt(x):
    i = plsc.bitcast(x, jnp.int32)
    i = jnp.int32(0x5F3759DF) - lax.shift_right_logical(i, 1)
    y = plsc.bitcast(i, jnp.float32)
    for _ in range(3):
        y = y * (1.5 - 0.5 * x * y * y)
    return y


def _maxnorm_scale(ss):
    n = ss * _rsqrt(ss)
    return jnp.minimum(1.0, 1.0 / (n + 1e-7))


def _iota16():
    return lax.iota(jnp.int32, 16)


def _bin_chunk(rows_i, rows_c, c0, c1, acc):
    zero = jnp.zeros((L,), jnp.float32)
    iot = _iota16()

    def group(g, acc):
        rowv = g * L + iot
        f = zero
        ss = zero
        for d in range(DIM):
            col = (iot + d) & (DIM - 1)
            iv = plsc.load_gather(rows_i, [rowv, col])
            cv = plsc.load_gather(rows_c, [rowv, col])
            f = f + iv * cv
            ss = ss + iv * iv
        fs = f * _maxnorm_scale(ss)
        return acc + jnp.maximum(c0 + c1 * fs, 0.0)

    return lax.fori_loop(0, G, group, acc)


def _tri_chunk(rows_h, rows_t, rows_r, c0, c1, acc):
    zero = jnp.zeros((L,), jnp.float32)
    iot = _iota16()

    def group(g, acc):
        rowv = g * L + iot
        f1 = zero
        f2 = zero
        sh = zero
        st = zero
        sr = zero
        for d in range(DIM):
            col = (iot + d) & (DIM - 1)
            col2 = col + DIM
            hv = plsc.load_gather(rows_h, [rowv, col])
            tv = plsc.load_gather(rows_t, [rowv, col])
            r1 = plsc.load_gather(rows_r, [rowv, col])
            r2 = plsc.load_gather(rows_r, [rowv, col2])
            f1 = f1 + hv * r1
            f2 = f2 + tv * r2
            sh = sh + hv * hv
            st = st + tv * tv
            sr = sr + r1 * r1 + r2 * r2
        fs = _maxnorm_scale(sr) * (_maxnorm_scale(sh) * f1
                                   + _maxnorm_scale(st) * f2)
        return acc + jnp.maximum(c0 + c1 * fs, 0.0)

    return lax.fori_loop(0, G, group, acc)


@functools.partial(
    pl.kernel,
    out_type=jax.ShapeDtypeStruct((NW, L), jnp.float32),
    mesh=plsc.VectorSubcoreMesh(core_axis_name="c", subcore_axis_name="s"),
    compiler_params=pltpu.CompilerParams(
        use_tc_tiling_on_sc=False, needs_layout_passes=False),
    scratch_types=[
        pltpu.VMEM((10 * BPW,), jnp.int32),
        pltpu.VMEM((C, DIM), jnp.float32),
        pltpu.VMEM((C, DIM), jnp.float32),
        pltpu.VMEM((C, 2 * DIM), jnp.float32),
        pltpu.VMEM((L,), jnp.float32),
        pltpu.SemaphoreType.DMA,
        pltpu.SemaphoreType.DMA,
        pltpu.SemaphoreType.DMA,
    ],
)
def _sc_loss(inst_pos, con_pos, inst_neg, con_neg, head_pos, tail_pos,
             rel_pos, head_neg, tail_neg, rel_neg,
             instance_emb, concept_emb, rel_emb, out_hbm,
             idx_all, rows_a, rows_b, rows_r, part_v, sem_a, sem_b, sem_r):
    wid = lax.axis_index("s") * NC + lax.axis_index("c")
    base = wid * BPW

    idx_arrays = (inst_pos, con_pos, inst_neg, con_neg, head_pos, tail_pos,
                  rel_pos, head_neg, tail_neg, rel_neg)
    for k, arr in enumerate(idx_arrays):
        pltpu.sync_copy(arr.at[pl.ds(base, BPW)],
                        idx_all.at[pl.ds(k * BPW, BPW)])

    def isl(k, ch):
        return idx_all.at[pl.ds(k * BPW + ch * C, C)]

    def chunk(ch, acc):
        cp_a = pltpu.async_copy(instance_emb.at[isl(0, ch)], rows_a, sem_a)
        cp_b = pltpu.async_copy(concept_emb.at[isl(1, ch)], rows_b, sem_b)
        cp_a.wait()
        cp_b.wait()
        acc = _bin_chunk(rows_a, rows_b, B_T, -1.0, acc)

        cp_a = pltpu.async_copy(instance_emb.at[isl(2, ch)], rows_a, sem_a)
        cp_b = pltpu.async_copy(concept_emb.at[isl(3, ch)], rows_b, sem_b)
        cp_a.wait()
        cp_b.wait()
        acc = _bin_chunk(rows_a, rows_b, -B_T, 1.0, acc)

        cp_a = pltpu.async_copy(instance_emb.at[isl(4, ch)], rows_a, sem_a)
        cp_b = pltpu.async_copy(instance_emb.at[isl(5, ch)], rows_b, sem_b)
        cp_r = pltpu.async_copy(rel_emb.at[isl(6, ch)], rows_r, sem_r)
        cp_a.wait()
        cp_b.wait()
        cp_r.wait()
        acc = _tri_chunk(rows_a, rows_b, rows_r, B_R, -1.0, acc)

        cp_a = pltpu.async_copy(instance_emb.at[isl(7, ch)], rows_a, sem_a)
        cp_b = pltpu.async_copy(instance_emb.at[isl(8, ch)], rows_b, sem_b)
        cp_r = pltpu.async_copy(rel_emb.at[isl(9, ch)], rows_r, sem_r)
        cp_a.wait()
        cp_b.wait()
        cp_r.wait()
        acc = _tri_chunk(rows_a, rows_b, rows_r, -B_R, 1.0, acc)
        return acc

    acc = lax.fori_loop(0, NCH, chunk, jnp.zeros((L,), jnp.float32))

    part_v[...] = acc
    pltpu.sync_copy(part_v, out_hbm.at[wid])


def _sum_body(x_ref, o_ref):
    o_ref[0, 0] = jnp.sum(x_ref[...])


def _final_sum(partials):
    return pl.pallas_call(
        _sum_body,
        out_shape=jax.ShapeDtypeStruct((1, 1), jnp.float32),
        in_specs=[pl.BlockSpec(memory_space=pltpu.VMEM)],
        out_specs=pl.BlockSpec(memory_space=pltpu.SMEM),
    )(partials)


def kernel(inst_pos, con_pos, inst_neg, con_neg, head_pos, tail_pos,
           rel_pos, head_neg, tail_neg, rel_neg,
           instance_emb, concept_emb, rel_emb):
    partials = _sc_loss(inst_pos, con_pos, inst_neg, con_neg,
                        head_pos, tail_pos, rel_pos,
                        head_neg, tail_neg, rel_neg,
                        instance_emb, concept_emb, rel_emb)
    return _final_sum(partials)[0, 0]


# speedup vs baseline: 1.3635x; 1.0827x over previous
import functools

import jax
import jax.numpy as jnp
from jax import lax
from jax.experimental import pallas as pl
from jax.experimental.pallas import tpu as pltpu
from jax.experimental.pallas import tpu_sc as plsc

DIM = 64
PD = 128
B = 16384
B_T = 1.0
B_R = 2.0

NC = 2
NS = 16
L = 16
NW = NC * NS
BPW = B // NW
C = 256
G = C // L
NCH = BPW // C


def _rsqrt(x):
    i = plsc.bitcast(x, jnp.int32)
    i = jnp.int32(0x5F3759DF) - lax.shift_right_logical(i, 1)
    y = plsc.bitcast(i, jnp.float32)
    for _ in range(3):
        y = y * (1.5 - 0.5 * x * y * y)
    return y


def _maxnorm_scale(ss):
    n = ss * _rsqrt(ss)
    return jnp.minimum(1.0, 1.0 / (n + 1e-7))


def _iota16():
    return lax.iota(jnp.int32, 16)


def _bin_chunk(rows_i, rows_c, c0, c1, acc):
    zero = jnp.zeros((L,), jnp.float32)
    iot = _iota16()

    def group(g, acc):
        rowv = g * L + iot
        f = zero
        ss = zero
        for d in range(DIM):
            col = (iot + d) & (DIM - 1)
            iv = plsc.load_gather(rows_i, [rowv, col])
            cv = plsc.load_gather(rows_c, [rowv, col])
            f = f + iv * cv
            ss = ss + iv * iv
        fs = f * _maxnorm_scale(ss)
        return acc + jnp.maximum(c0 + c1 * fs, 0.0)

    return lax.fori_loop(0, G, group, acc)


def _tri_chunk(rows_h, rows_t, rows_r, c0, c1, acc):
    zero = jnp.zeros((L,), jnp.float32)
    iot = _iota16()

    def group(g, acc):
        rowv = g * L + iot
        f1 = zero
        f2 = zero
        sh = zero
        st = zero
        sr = zero
        for d in range(DIM):
            col = (iot + d) & (DIM - 1)
            col2 = col + DIM
            hv = plsc.load_gather(rows_h, [rowv, col])
            tv = plsc.load_gather(rows_t, [rowv, col])
            r1 = plsc.load_gather(rows_r, [rowv, col])
            r2 = plsc.load_gather(rows_r, [rowv, col2])
            f1 = f1 + hv * r1
            f2 = f2 + tv * r2
            sh = sh + hv * hv
            st = st + tv * tv
            sr = sr + r1 * r1 + r2 * r2
        fs = _maxnorm_scale(sr) * (_maxnorm_scale(sh) * f1
                                   + _maxnorm_scale(st) * f2)
        return acc + jnp.maximum(c0 + c1 * fs, 0.0)

    return lax.fori_loop(0, G, group, acc)


@functools.partial(
    pl.kernel,
    out_type=jax.ShapeDtypeStruct((NW, L), jnp.float32),
    mesh=plsc.VectorSubcoreMesh(core_axis_name="c", subcore_axis_name="s"),
    compiler_params=pltpu.CompilerParams(
        use_tc_tiling_on_sc=False, needs_layout_passes=False),
    scratch_types=[
        pltpu.VMEM((10 * BPW,), jnp.int32),
        pltpu.VMEM((C, PD), jnp.float32),
        pltpu.VMEM((C, PD), jnp.float32),
        pltpu.VMEM((C, PD), jnp.float32),
        pltpu.VMEM((L,), jnp.float32),
        pltpu.SemaphoreType.DMA,
        pltpu.SemaphoreType.DMA,
        pltpu.SemaphoreType.DMA,
    ],
)
def _sc_loss(inst_pos, con_pos, inst_neg, con_neg, head_pos, tail_pos,
             rel_pos, head_neg, tail_neg, rel_neg,
             inst_tab, con_tab, rel_tab, out_hbm,
             idx_all, rows_a, rows_b, rows_r, part_v, sem_a, sem_b, sem_r):
    wid = lax.axis_index("s") * NC + lax.axis_index("c")
    base = wid * BPW

    idx_arrays = (inst_pos, con_pos, inst_neg, con_neg, head_pos, tail_pos,
                  rel_pos, head_neg, tail_neg, rel_neg)
    cps = [pltpu.async_copy(arr.at[pl.ds(base, BPW)],
                            idx_all.at[pl.ds(k * BPW, BPW)], sem_a)
           for k, arr in enumerate(idx_arrays)]
    for cp in cps:
        cp.wait()

    def isl(k, ch):
        return idx_all.at[pl.ds(k * BPW + ch * C, C)]

    def chunk(ch, acc):
        cp_a = pltpu.async_copy(inst_tab.at[isl(0, ch)], rows_a, sem_a)
        cp_b = pltpu.async_copy(con_tab.at[isl(1, ch)], rows_b, sem_b)
        cp_a.wait()
        cp_b.wait()
        acc = _bin_chunk(rows_a, rows_b, B_T, -1.0, acc)

        cp_a = pltpu.async_copy(inst_tab.at[isl(2, ch)], rows_a, sem_a)
        cp_b = pltpu.async_copy(con_tab.at[isl(3, ch)], rows_b, sem_b)
        cp_a.wait()
        cp_b.wait()
        acc = _bin_chunk(rows_a, rows_b, -B_T, 1.0, acc)

        cp_a = pltpu.async_copy(inst_tab.at[isl(4, ch)], rows_a, sem_a)
        cp_b = pltpu.async_copy(inst_tab.at[isl(5, ch)], rows_b, sem_b)
        cp_r = pltpu.async_copy(rel_tab.at[isl(6, ch)], rows_r, sem_r)
        cp_a.wait()
        cp_b.wait()
        cp_r.wait()
        acc = _tri_chunk(rows_a, rows_b, rows_r, B_R, -1.0, acc)

        cp_a = pltpu.async_copy(inst_tab.at[isl(7, ch)], rows_a, sem_a)
        cp_b = pltpu.async_copy(inst_tab.at[isl(8, ch)], rows_b, sem_b)
        cp_r = pltpu.async_copy(rel_tab.at[isl(9, ch)], rows_r, sem_r)
        cp_a.wait()
        cp_b.wait()
        cp_r.wait()
        acc = _tri_chunk(rows_a, rows_b, rows_r, -B_R, 1.0, acc)
        return acc

    acc = lax.fori_loop(0, NCH, chunk, jnp.zeros((L,), jnp.float32))

    part_v[...] = acc
    pltpu.sync_copy(part_v, out_hbm.at[wid])


def _sum_body(x_ref, o_ref):
    o_ref[0, 0] = jnp.sum(x_ref[...])


def _final_sum(partials):
    return pl.pallas_call(
        _sum_body,
        out_shape=jax.ShapeDtypeStruct((1, 1), jnp.float32),
        in_specs=[pl.BlockSpec(memory_space=pltpu.VMEM)],
        out_specs=pl.BlockSpec(memory_space=pltpu.SMEM),
    )(partials)


def kernel(inst_pos, con_pos, inst_neg, con_neg, head_pos, tail_pos,
           rel_pos, head_neg, tail_neg, rel_neg,
           instance_emb, concept_emb, rel_emb):
    inst_tab = jnp.pad(instance_emb, ((0, 0), (0, PD - DIM)))
    con_tab = jnp.pad(concept_emb, ((0, 0), (0, PD - DIM)))
    partials = _sc_loss(inst_pos, con_pos, inst_neg, con_neg,
                        head_pos, tail_pos, rel_pos,
                        head_neg, tail_neg, rel_neg,
                        inst_tab, con_tab, rel_emb)
    return _final_sum(partials)[0, 0]


# speedup vs baseline: 1.3813x; 1.0130x over previous
import functools

import jax
import jax.numpy as jnp
from jax import lax
from jax.experimental import pallas as pl
from jax.experimental.pallas import tpu as pltpu
from jax.experimental.pallas import tpu_sc as plsc

DIM = 64
PD = 128
B = 16384
B_T = 1.0
B_R = 2.0

NC = 2
NS = 16
L = 16
NW = NC * NS
BPW = B // NW
C = 256
G = C // L
NCH = BPW // C


def _rsqrt(x):
    i = plsc.bitcast(x, jnp.int32)
    i = jnp.int32(0x5F3759DF) - lax.shift_right_logical(i, 1)
    y = plsc.bitcast(i, jnp.float32)
    for _ in range(3):
        y = y * (1.5 - 0.5 * x * y * y)
    return y


def _maxnorm_scale(ss):
    n = ss * _rsqrt(ss)
    return jnp.minimum(1.0, 1.0 / (n + 1e-7))


def _iota16():
    return lax.iota(jnp.int32, 16)


def _bin_chunk(rows_i, rows_c, c0, c1, acc):
    zero = jnp.zeros((L,), jnp.float32)
    iot = _iota16()

    def group(g, acc):
        rowv = g * L + iot
        f = zero
        ss = zero
        for d in range(DIM):
            col = (iot + d) & (DIM - 1)
            iv = plsc.load_gather(rows_i, [rowv, col])
            cv = plsc.load_gather(rows_c, [rowv, col])
            f = f + iv * cv
            ss = ss + iv * iv
        fs = f * _maxnorm_scale(ss)
        return acc + jnp.maximum(c0 + c1 * fs, 0.0)

    return lax.fori_loop(0, G, group, acc)


def _tri_chunk(rows_h, rows_t, rows_r, c0, c1, acc):
    zero = jnp.zeros((L,), jnp.float32)
    iot = _iota16()

    def group(g, acc):
        rowv = g * L + iot
        f1 = zero
        f2 = zero
        sh = zero
        st = zero
        sr = zero
        for d in range(DIM):
            col = (iot + d) & (DIM - 1)
            col2 = col + DIM
            hv = plsc.load_gather(rows_h, [rowv, col])
            tv = plsc.load_gather(rows_t, [rowv, col])
            r1 = plsc.load_gather(rows_r, [rowv, col])
            r2 = plsc.load_gather(rows_r, [rowv, col2])
            f1 = f1 + hv * r1
            f2 = f2 + tv * r2
            sh = sh + hv * hv
            st = st + tv * tv
            sr = sr + r1 * r1 + r2 * r2
        fs = _maxnorm_scale(sr) * (_maxnorm_scale(sh) * f1
                                   + _maxnorm_scale(st) * f2)
        return acc + jnp.maximum(c0 + c1 * fs, 0.0)

    return lax.fori_loop(0, G, group, acc)


@functools.partial(
    pl.kernel,
    out_type=jax.ShapeDtypeStruct((NW, L), jnp.float32),
    mesh=plsc.VectorSubcoreMesh(core_axis_name="c", subcore_axis_name="s"),
    compiler_params=pltpu.CompilerParams(
        use_tc_tiling_on_sc=False, needs_layout_passes=False),
    scratch_types=[
        pltpu.VMEM((10 * BPW,), jnp.int32),
        pltpu.VMEM((C, DIM), jnp.float32),
        pltpu.VMEM((C, DIM), jnp.float32),
        pltpu.VMEM((C, PD), jnp.float32),
        pltpu.VMEM((L,), jnp.float32),
        pltpu.SemaphoreType.DMA,
        pltpu.SemaphoreType.DMA,
        pltpu.SemaphoreType.DMA,
    ],
)
def _sc_loss(inst_pos, con_pos, inst_neg, con_neg, head_pos, tail_pos,
             rel_pos, head_neg, tail_neg, rel_neg,
             inst_tab, con_tab, rel_tab, out_hbm,
             idx_all, rows_a, rows_b, rows_r, part_v, sem_a, sem_b, sem_r):
    wid = lax.axis_index("s") * NC + lax.axis_index("c")
    base = wid * BPW

    idx_arrays = (inst_pos, con_pos, inst_neg, con_neg, head_pos, tail_pos,
                  rel_pos, head_neg, tail_neg, rel_neg)
    cps = [pltpu.async_copy(arr.at[pl.ds(base, BPW)],
                            idx_all.at[pl.ds(k * BPW, BPW)], sem_a)
           for k, arr in enumerate(idx_arrays)]
    for cp in cps:
        cp.wait()

    for k in (0, 1, 2, 3, 4, 5, 7, 8):

        def dbl(i, _, k=k):
            off = k * BPW + i * L
            idx_all[pl.ds(off, L)] = lax.shift_left(
                idx_all[pl.ds(off, L)], 1)
            return 0

        lax.fori_loop(0, BPW // L, dbl, 0)

    def isl(k, ch):
        return idx_all.at[pl.ds(k * BPW + ch * C, C)]

    def chunk(ch, acc):
        cp_a = pltpu.async_copy(inst_tab.at[isl(0, ch)], rows_a, sem_a)
        cp_b = pltpu.async_copy(con_tab.at[isl(1, ch)], rows_b, sem_b)
        cp_a.wait()
        cp_b.wait()
        acc = _bin_chunk(rows_a, rows_b, B_T, -1.0, acc)

        cp_a = pltpu.async_copy(inst_tab.at[isl(2, ch)], rows_a, sem_a)
        cp_b = pltpu.async_copy(con_tab.at[isl(3, ch)], rows_b, sem_b)
        cp_a.wait()
        cp_b.wait()
        acc = _bin_chunk(rows_a, rows_b, -B_T, 1.0, acc)

        cp_a = pltpu.async_copy(inst_tab.at[isl(4, ch)], rows_a, sem_a)
        cp_b = pltpu.async_copy(inst_tab.at[isl(5, ch)], rows_b, sem_b)
        cp_r = pltpu.async_copy(rel_tab.at[isl(6, ch)], rows_r, sem_r)
        cp_a.wait()
        cp_b.wait()
        cp_r.wait()
        acc = _tri_chunk(rows_a, rows_b, rows_r, B_R, -1.0, acc)

        cp_a = pltpu.async_copy(inst_tab.at[isl(7, ch)], rows_a, sem_a)
        cp_b = pltpu.async_copy(inst_tab.at[isl(8, ch)], rows_b, sem_b)
        cp_r = pltpu.async_copy(rel_tab.at[isl(9, ch)], rows_r, sem_r)
        cp_a.wait()
        cp_b.wait()
        cp_r.wait()
        acc = _tri_chunk(rows_a, rows_b, rows_r, -B_R, 1.0, acc)
        return acc

    acc = lax.fori_loop(0, NCH, chunk, jnp.zeros((L,), jnp.float32))

    part_v[...] = acc
    pltpu.sync_copy(part_v, out_hbm.at[wid])


def _sum_body(x_ref, o_ref):
    o_ref[0, 0] = jnp.sum(x_ref[...])


def _final_sum(partials):
    return pl.pallas_call(
        _sum_body,
        out_shape=jax.ShapeDtypeStruct((1, 1), jnp.float32),
        in_specs=[pl.BlockSpec(memory_space=pltpu.VMEM)],
        out_specs=pl.BlockSpec(memory_space=pltpu.SMEM),
    )(partials)


def kernel(inst_pos, con_pos, inst_neg, con_neg, head_pos, tail_pos,
           rel_pos, head_neg, tail_neg, rel_neg,
           instance_emb, concept_emb, rel_emb):
    inst_tab = jnp.pad(instance_emb, ((0, 0), (0, PD - DIM)))
    inst_tab = inst_tab.reshape(2 * instance_emb.shape[0], DIM)
    con_tab = jnp.pad(concept_emb, ((0, 0), (0, PD - DIM)))
    con_tab = con_tab.reshape(2 * concept_emb.shape[0], DIM)
    partials = _sc_loss(inst_pos, con_pos, inst_neg, con_neg,
                        head_pos, tail_pos, rel_pos,
                        head_neg, tail_neg, rel_neg,
                        inst_tab, con_tab, rel_emb)
    return _final_sum(partials)[0, 0]


# speedup vs baseline: 1.3957x; 1.0104x over previous
import functools

import jax
import jax.numpy as jnp
from jax import lax
from jax.experimental import pallas as pl
from jax.experimental.pallas import tpu as pltpu
from jax.experimental.pallas import tpu_sc as plsc

DIM = 64
PD = 128
B = 16384
B_T = 1.0
B_R = 2.0

NC = 2
NS = 16
L = 16
NW = NC * NS
BPW = B // NW
C = 256
G = C // L
NCH = BPW // C


def _rsqrt(x):
    i = plsc.bitcast(x, jnp.int32)
    i = jnp.int32(0x5F3759DF) - lax.shift_right_logical(i, 1)
    y = plsc.bitcast(i, jnp.float32)
    for _ in range(3):
        y = y * (1.5 - 0.5 * x * y * y)
    return y


def _maxnorm_scale(ss):
    n = ss * _rsqrt(ss)
    return jnp.minimum(1.0, 1.0 / (n + 1e-7))


def _iota16():
    return lax.iota(jnp.int32, 16)


def _bin_chunk(rows_i, rows_c, c0, c1, acc):
    zero = jnp.zeros((L,), jnp.float32)
    iot = _iota16()

    def group(g, acc):
        rowv = g * L + iot
        f = zero
        ss = zero
        for d in range(DIM):
            col = (iot + d) & (DIM - 1)
            iv = plsc.load_gather(rows_i, [rowv, col])
            cv = plsc.load_gather(rows_c, [rowv, col])
            f = f + iv * cv
            ss = ss + iv * iv
        fs = f * _maxnorm_scale(ss)
        return acc + jnp.maximum(c0 + c1 * fs, 0.0)

    return lax.fori_loop(0, G, group, acc)


def _tri_chunk(rows_h, rows_t, rows_r, c0, c1, acc):
    zero = jnp.zeros((L,), jnp.float32)
    iot = _iota16()

    def group(g, acc):
        rowv = g * L + iot
        f1 = zero
        f2 = zero
        sh = zero
        st = zero
        sr = zero
        for d in range(DIM):
            col = (iot + d) & (DIM - 1)
            col2 = col + DIM
            hv = plsc.load_gather(rows_h, [rowv, col])
            tv = plsc.load_gather(rows_t, [rowv, col])
            r1 = plsc.load_gather(rows_r, [rowv, col])
            r2 = plsc.load_gather(rows_r, [rowv, col2])
            f1 = f1 + hv * r1
            f2 = f2 + tv * r2
            sh = sh + hv * hv
            st = st + tv * tv
            sr = sr + r1 * r1 + r2 * r2
        fs = _maxnorm_scale(sr) * (_maxnorm_scale(sh) * f1
                                   + _maxnorm_scale(st) * f2)
        return acc + jnp.maximum(c0 + c1 * fs, 0.0)

    return lax.fori_loop(0, G, group, acc)


@functools.partial(
    pl.kernel,
    out_type=jax.ShapeDtypeStruct((NW, L), jnp.float32),
    mesh=plsc.VectorSubcoreMesh(core_axis_name="c", subcore_axis_name="s"),
    compiler_params=pltpu.CompilerParams(
        use_tc_tiling_on_sc=False, needs_layout_passes=False),
    scratch_types=[
        pltpu.VMEM((10 * BPW,), jnp.int32),
        pltpu.VMEM((C, DIM), jnp.float32),
        pltpu.VMEM((C, DIM), jnp.float32),
        pltpu.VMEM((C, DIM), jnp.float32),
        pltpu.VMEM((C, DIM), jnp.float32),
        pltpu.VMEM((C, PD), jnp.float32),
        pltpu.VMEM((L,), jnp.float32),
        pltpu.SemaphoreType.DMA,
        pltpu.SemaphoreType.DMA,
        pltpu.SemaphoreType.DMA,
    ],
)
def _sc_loss(inst_pos, con_pos, inst_neg, con_neg, head_pos, tail_pos,
             rel_pos, head_neg, tail_neg, rel_neg,
             inst_tab, con_tab, rel_tab, out_hbm,
             idx_all, a0, b0, a1, b1, rows_r, part_v, sem_a, sem_b, sem_r):
    wid = lax.axis_index("s") * NC + lax.axis_index("c")
    base = wid * BPW

    idx_arrays = (inst_pos, con_pos, inst_neg, con_neg, head_pos, tail_pos,
                  rel_pos, head_neg, tail_neg, rel_neg)
    cps = [pltpu.async_copy(arr.at[pl.ds(base, BPW)],
                            idx_all.at[pl.ds(k * BPW, BPW)], sem_a)
           for k, arr in enumerate(idx_arrays)]
    for cp in cps:
        cp.wait()

    for k in (0, 1, 2, 3, 4, 5, 7, 8):

        def dbl(i, _, k=k):
            off = k * BPW + i * L
            idx_all[pl.ds(off, L)] = lax.shift_left(
                idx_all[pl.ds(off, L)], 1)
            return 0

        lax.fori_loop(0, BPW // L, dbl, 0)

    def isl(k, ch):
        return idx_all.at[pl.ds(k * BPW + ch * C, C)]

    def chunk(ch, acc):
        cpa0 = pltpu.async_copy(inst_tab.at[isl(0, ch)], a0, sem_a)
        cpb0 = pltpu.async_copy(con_tab.at[isl(1, ch)], b0, sem_b)
        cpa1 = pltpu.async_copy(inst_tab.at[isl(2, ch)], a1, sem_a)
        cpb1 = pltpu.async_copy(con_tab.at[isl(3, ch)], b1, sem_b)
        cpa0.wait()
        cpb0.wait()
        acc = _bin_chunk(a0, b0, B_T, -1.0, acc)

        cph0 = pltpu.async_copy(inst_tab.at[isl(4, ch)], a0, sem_a)
        cpt0 = pltpu.async_copy(inst_tab.at[isl(5, ch)], b0, sem_b)
        cpr0 = pltpu.async_copy(rel_tab.at[isl(6, ch)], rows_r, sem_r)
        cpa1.wait()
        cpb1.wait()
        acc = _bin_chunk(a1, b1, -B_T, 1.0, acc)

        cph1 = pltpu.async_copy(inst_tab.at[isl(7, ch)], a1, sem_a)
        cpt1 = pltpu.async_copy(inst_tab.at[isl(8, ch)], b1, sem_b)
        cph0.wait()
        cpt0.wait()
        cpr0.wait()
        acc = _tri_chunk(a0, b0, rows_r, B_R, -1.0, acc)

        cpr1 = pltpu.async_copy(rel_tab.at[isl(9, ch)], rows_r, sem_r)
        cph1.wait()
        cpt1.wait()
        cpr1.wait()
        acc = _tri_chunk(a1, b1, rows_r, -B_R, 1.0, acc)
        return acc

    acc = lax.fori_loop(0, NCH, chunk, jnp.zeros((L,), jnp.float32))

    part_v[...] = acc
    pltpu.sync_copy(part_v, out_hbm.at[wid])


def _sum_body(x_ref, o_ref):
    o_ref[0, 0] = jnp.sum(x_ref[...])


def _final_sum(partials):
    return pl.pallas_call(
        _sum_body,
        out_shape=jax.ShapeDtypeStruct((1, 1), jnp.float32),
        in_specs=[pl.BlockSpec(memory_space=pltpu.VMEM)],
        out_specs=pl.BlockSpec(memory_space=pltpu.SMEM),
    )(partials)


def kernel(inst_pos, con_pos, inst_neg, con_neg, head_pos, tail_pos,
           rel_pos, head_neg, tail_neg, rel_neg,
           instance_emb, concept_emb, rel_emb):
    inst_tab = jnp.pad(instance_emb, ((0, 0), (0, PD - DIM)))
    inst_tab = inst_tab.reshape(2 * instance_emb.shape[0], DIM)
    con_tab = jnp.pad(concept_emb, ((0, 0), (0, PD - DIM)))
    con_tab = con_tab.reshape(2 * concept_emb.shape[0], DIM)
    partials = _sc_loss(inst_pos, con_pos, inst_neg, con_neg,
                        head_pos, tail_pos, rel_pos,
                        head_neg, tail_neg, rel_neg,
                        inst_tab, con_tab, rel_emb)
    return _final_sum(partials)[0, 0]
